# baseline (speedup 1.0000x reference)
import functools

import jax
import jax.numpy as jnp
from jax import lax
from jax.experimental import pallas as pl
from jax.experimental.pallas import tpu as pltpu
from jax.experimental.pallas import tpu_sc as plsc

HID = 64
BN = 128
NW = 32
IC = 128


def _sc_gather_body(rows_per_w, table, idxh, idxt, outh, outt,
                    idxh_v, idxt_v, rowsh_v, rowst_v, sem):
    wid = lax.axis_index("s") * 2 + lax.axis_index("c")
    nchunks = rows_per_w // IC
    pltpu.sync_copy(idxh.at[wid], idxh_v)
    pltpu.sync_copy(idxt.at[wid], idxt_v)
    copies = []
    for j in range(nchunks):
        copies.append(pltpu.async_copy(
            table.at[idxh_v.at[j]], rowsh_v.at[pl.ds(j * IC, IC)], sem))
        copies.append(pltpu.async_copy(
            table.at[idxt_v.at[j]], rowst_v.at[pl.ds(j * IC, IC)], sem))
    for c in copies:
        c.wait()
    base = wid * rows_per_w
    pltpu.sync_copy(rowsh_v, outh.at[pl.ds(base, rows_per_w)])
    pltpu.sync_copy(rowst_v, outt.at[pl.ds(base, rows_per_w)])


def _sc_gather(table, h_idx, t_idx):
    b = h_idx.shape[0]
    rows_per_w = b // NW
    nchunks = rows_per_w // IC
    mesh = plsc.VectorSubcoreMesh(core_axis_name="c", subcore_axis_name="s")
    f = pl.kernel(
        functools.partial(_sc_gather_body, rows_per_w),
        mesh=mesh,
        out_type=(
            jax.ShapeDtypeStruct((b, HID), jnp.float32),
            jax.ShapeDtypeStruct((b, HID), jnp.float32),
        ),
        scratch_types=[
            pltpu.VMEM((nchunks, IC), jnp.int32),
            pltpu.VMEM((nchunks, IC), jnp.int32),
            pltpu.VMEM((rows_per_w, HID), jnp.float32),
            pltpu.VMEM((rows_per_w, HID), jnp.float32),
            pltpu.SemaphoreType.DMA,
        ],
    )
    return f(table,
             h_idx.reshape(NW, nchunks, IC),
             t_idx.reshape(NW, nchunks, IC))


UNROLL = 8


def _tc_body(r_ref, nxt_ref, hgt_ref, tgt_ref, rel_ref, out_ref, acc_ref):
    base = pl.program_id(0) * BN
    tbt = tgt_ref[...].astype(jnp.bfloat16)
    acc_ref[...] = jnp.zeros((HID, BN), jnp.float32)
    lane_iota = lax.broadcasted_iota(jnp.int32, (HID, BN), 1)

    def cond(i):
        return i < BN

    def body(i):
        starts, ends = [], []
        cur = i
        for _ in range(UNROLL):
            s = jnp.minimum(cur, BN - 1)
            e = jnp.minimum(nxt_ref[s] - base, BN)
            starts.append((s, cur))
            ends.append(e)
            cur = e
        uts = []
        for (s, raw_s), e in zip(starts, ends):
            k = r_ref[s]
            rk = rel_ref[k].astype(jnp.bfloat16)
            ut = lax.dot_general(rk, tbt, (((1,), (0,)), ((), ())),
                                 preferred_element_type=jnp.float32)
            mask = (lane_iota >= raw_s) & (lane_iota < e)
            uts.append(jnp.where(mask, ut, 0.0))
        while len(uts) > 1:
            uts = [a + b for a, b in zip(uts[::2], uts[1::2])]
        span = (lane_iota >= i) & (lane_iota < cur)
        acc_ref[...] = jnp.where(span, uts[0], acc_ref[...])
        return cur

    lax.while_loop(cond, body, jnp.int32(0))
    out_ref[...] = -jnp.sum(hgt_ref[...] * acc_ref[...], axis=0, keepdims=True)


def _tc_scores(r_s, nxt, hgt, tgt, rel3):
    b = r_s.shape[0]
    nr = rel3.shape[0]
    return pl.pallas_call(
        _tc_body,
        grid=(b // BN,),
        in_specs=[
            pl.BlockSpec((BN,), lambda i: (i,), memory_space=pltpu.SMEM),
            pl.BlockSpec((BN,), lambda i: (i,), memory_space=pltpu.SMEM),
            pl.BlockSpec((HID, BN), lambda i: (0, i)),
            pl.BlockSpec((HID, BN), lambda i: (0, i)),
            pl.BlockSpec((nr, HID, HID), lambda i: (0, 0, 0)),
        ],
        out_specs=pl.BlockSpec((1, BN), lambda i: (0, i)),
        out_shape=jax.ShapeDtypeStruct((1, b), jnp.float32),
        scratch_shapes=[pltpu.VMEM((HID, BN), jnp.float32)],
    )(r_s, nxt, hgt, tgt, rel3)


def kernel(predict_h, predict_t, predict_r, ent_embeddings, rel_matrices):
    b = predict_h.shape[0]
    nr = rel_matrices.shape[0]
    shift = (b - 1).bit_length()

    r32 = predict_r.astype(jnp.int32)
    h32 = predict_h.astype(jnp.int32)
    t32 = predict_t.astype(jnp.int32)

    pos = lax.iota(jnp.int32, b)
    key = (r32 << shift) | pos
    key_s, h_idx, t_idx = lax.sort((key, h32, t32), num_keys=1)
    r_s = key_s >> shift
    order = key_s & (b - 1)

    boundary = jnp.concatenate(
        [r_s[1:] != r_s[:-1], jnp.ones((1,), dtype=bool)])
    end_pos = jnp.where(boundary, pos + 1, jnp.int32(b))
    nxt = lax.cummin(end_pos, axis=0, reverse=True)

    ent_t = ent_embeddings.T
    hgt = jnp.take(ent_t, h_idx, axis=1, mode="clip")
    tgt = jnp.take(ent_t, t_idx, axis=1, mode="clip")
    rel3 = rel_matrices.reshape(nr, HID, HID)
    ssort = _tc_scores(r_s, nxt, hgt, tgt, rel3)

    _, out_flat = lax.sort((order, ssort[0]), num_keys=1)
    return out_flat[:, None]


# speedup vs baseline: 1.0245x; 1.0245x over previous
import functools

import jax
import jax.numpy as jnp
from jax import lax
from jax.experimental import pallas as pl
from jax.experimental.pallas import tpu as pltpu
from jax.experimental.pallas import tpu_sc as plsc

HID = 64
BN = 256
NW = 32
IC = 128


def _sc_gather_body(rows_per_w, table, idxh, idxt, outh, outt,
                    idxh_v, idxt_v, rowsh_v, rowst_v, sem):
    wid = lax.axis_index("s") * 2 + lax.axis_index("c")
    nchunks = rows_per_w // IC
    pltpu.sync_copy(idxh.at[wid], idxh_v)
    pltpu.sync_copy(idxt.at[wid], idxt_v)
    copies = []
    for j in range(nchunks):
        copies.append(pltpu.async_copy(
            table.at[idxh_v.at[j]], rowsh_v.at[pl.ds(j * IC, IC)], sem))
        copies.append(pltpu.async_copy(
            table.at[idxt_v.at[j]], rowst_v.at[pl.ds(j * IC, IC)], sem))
    for c in copies:
        c.wait()
    base = wid * rows_per_w
    pltpu.sync_copy(rowsh_v, outh.at[pl.ds(base, rows_per_w)])
    pltpu.sync_copy(rowst_v, outt.at[pl.ds(base, rows_per_w)])


def _sc_gather(table, h_idx, t_idx):
    b = h_idx.shape[0]
    rows_per_w = b // NW
    nchunks = rows_per_w // IC
    mesh = plsc.VectorSubcoreMesh(core_axis_name="c", subcore_axis_name="s")
    f = pl.kernel(
        functools.partial(_sc_gather_body, rows_per_w),
        mesh=mesh,
        out_type=(
            jax.ShapeDtypeStruct((b, HID), jnp.float32),
            jax.ShapeDtypeStruct((b, HID), jnp.float32),
        ),
        scratch_types=[
            pltpu.VMEM((nchunks, IC), jnp.int32),
            pltpu.VMEM((nchunks, IC), jnp.int32),
            pltpu.VMEM((rows_per_w, HID), jnp.float32),
            pltpu.VMEM((rows_per_w, HID), jnp.float32),
            pltpu.SemaphoreType.DMA,
        ],
    )
    return f(table,
             h_idx.reshape(NW, nchunks, IC),
             t_idx.reshape(NW, nchunks, IC))


UNROLL = 8


def _tc_body(r_ref, nxt_ref, hg_ref, tg_ref, rel_ref, out_ref, acc_ref):
    base = pl.program_id(0) * BN
    tb = tg_ref[...]
    row_iota = lax.broadcasted_iota(jnp.int32, (BN, HID), 0)

    def cond(i):
        return i < BN

    def body(i):
        starts, ends = [], []
        cur = i
        for _ in range(UNROLL):
            s = jnp.minimum(cur, BN - 1)
            e = jnp.minimum(nxt_ref[base + s] - base, BN)
            starts.append((s, cur))
            ends.append(e)
            cur = e
        uts = []
        for (s, raw_s), e in zip(starts, ends):
            k = r_ref[base + s]
            rk = rel_ref[k]
            u = lax.dot_general(tb, rk, (((1,), (1,)), ((), ())),
                                preferred_element_type=jnp.float32)
            mask = (row_iota >= raw_s) & (row_iota < e)
            uts.append(jnp.where(mask, u, 0.0))
        while len(uts) > 1:
            uts = [a + b for a, b in zip(uts[::2], uts[1::2])]
        span = (row_iota >= i) & (row_iota < cur)
        acc_ref[...] = jnp.where(span, uts[0], acc_ref[...])
        return cur

    lax.while_loop(cond, body, jnp.int32(0))
    out_ref[...] = -jnp.sum(hg_ref[...] * acc_ref[...], axis=1, keepdims=True)


def _tc_scores(r_s, nxt, hg, tg, rel3):
    b = r_s.shape[0]
    nr = rel3.shape[0]
    grid_spec = pltpu.PrefetchScalarGridSpec(
        num_scalar_prefetch=2,
        grid=(b // BN,),
        in_specs=[
            pl.BlockSpec((BN, HID), lambda i, r, nx: (i, 0)),
            pl.BlockSpec((BN, HID), lambda i, r, nx: (i, 0)),
            pl.BlockSpec((nr, HID, HID), lambda i, r, nx: (0, 0, 0)),
        ],
        out_specs=pl.BlockSpec((BN, 1), lambda i, r, nx: (i, 0)),
        scratch_shapes=[pltpu.VMEM((BN, HID), jnp.float32)],
    )
    return pl.pallas_call(
        _tc_body,
        grid_spec=grid_spec,
        out_shape=jax.ShapeDtypeStruct((b, 1), jnp.float32),
    )(r_s, nxt, hg, tg, rel3)


def kernel(predict_h, predict_t, predict_r, ent_embeddings, rel_matrices):
    b = predict_h.shape[0]
    nr = rel_matrices.shape[0]
    shift = (b - 1).bit_length()

    r32 = predict_r.astype(jnp.int32)
    h32 = predict_h.astype(jnp.int32)
    t32 = predict_t.astype(jnp.int32)

    pos = lax.iota(jnp.int32, b)
    key = (r32 << shift) | pos
    key_s, h_idx, t_idx = lax.sort((key, h32, t32), num_keys=1)
    r_s = key_s >> shift
    order = key_s & (b - 1)

    boundary = jnp.concatenate(
        [r_s[1:] != r_s[:-1], jnp.ones((1,), dtype=bool)])
    end_pos = jnp.where(boundary, pos + 1, jnp.int32(b))
    nxt = lax.cummin(end_pos, axis=0, reverse=True)

    hg = jnp.take(ent_embeddings, h_idx, axis=0, mode="clip")
    tg = jnp.take(ent_embeddings, t_idx, axis=0, mode="clip")
    rel3 = rel_matrices.reshape(nr, HID, HID)
    ssort = _tc_scores(r_s, nxt, hg, tg, rel3)

    _, out_flat = lax.sort((order, ssort[:, 0]), num_keys=1)
    return out_flat[:, None]
